# Initial kernel scaffold; baseline (speedup 1.0000x reference)
import jax
import jax.numpy as jnp
from jax.experimental import pallas as pl


def kernel(x, table):
    raise NotImplementedError("write your pallas kernel here")


# speedup vs baseline: 1.5238x; 1.5238x over previous
import functools

import jax
import jax.numpy as jnp
import numpy as np
from jax import lax
from jax.experimental import pallas as pl
from jax.experimental.pallas import tpu as pltpu
from jax.experimental.pallas import tpu_sc as plsc

_info = plsc.get_sparse_core_info()
_NC, _NS, _L = _info.num_cores, _info.num_subcores, _info.num_lanes
_NW = _NC * _NS
_B = 16384 * 4
_BPW = _B // _NW
_D = 10
_OPW = _BPW * _D
_PERIOD = 5
_XSTEP = _PERIOD * _L // _D
_NMACRO = _OPW // (_PERIOD * _L)

_mesh = plsc.VectorSubcoreMesh(core_axis_name="c", subcore_axis_name="s")

_PAT = np.asarray(
    [[(i + 16 * g) // _D for i in range(_L)] for g in range(_PERIOD)]
    + [[(i + 16 * g) % _D for i in range(_L)] for g in range(_PERIOD)],
    dtype=np.int32)


@functools.partial(
    pl.kernel,
    mesh=_mesh,
    compiler_params=pltpu.CompilerParams(needs_layout_passes=False),
    out_type=jax.ShapeDtypeStruct((_B * _D,), jnp.float32),
    scratch_types=[
        pltpu.VMEM((_BPW,), jnp.int32),
        pltpu.VMEM((100 * _D,), jnp.float32),
        pltpu.VMEM((_OPW,), jnp.float32),
        pltpu.VMEM((2 * _PERIOD, _L), jnp.int32),
    ],
)
def _gather_kernel(x_hbm, table_hbm, pat_hbm, out_hbm, idx_v, table_v,
                   out_v, pat_v):
    wid = lax.axis_index("s") * _NC + lax.axis_index("c")
    base = wid * _BPW
    pltpu.sync_copy(x_hbm.at[pl.ds(base, _BPW)], idx_v)
    pltpu.sync_copy(table_hbm, table_v)
    pltpu.sync_copy(pat_hbm, pat_v)

    qs = [pat_v[g] for g in range(_PERIOD)]
    rs = [pat_v[_PERIOD + g] for g in range(_PERIOD)]

    def body(m, _):
        xoff = m * _XSTEP
        ooff = m * (_PERIOD * _L)
        for g in range(_PERIOD):
            rows = plsc.load_gather(idx_v, [qs[g] + xoff])
            vals = plsc.load_gather(table_v, [rows * _D + rs[g]])
            out_v[pl.ds(ooff + 16 * g, _L)] = vals
        return 0

    lax.fori_loop(0, _NMACRO, body, 0)
    pltpu.sync_copy(out_v, out_hbm.at[pl.ds(wid * _OPW, _OPW)])


def kernel(x, table):
    out = _gather_kernel(x.reshape(-1), table.reshape(-1),
                         jnp.asarray(_PAT))
    return out.reshape(-1, 2, 2, 10)


# speedup vs baseline: 1.6365x; 1.0740x over previous
import functools

import jax
import jax.numpy as jnp
import numpy as np
from jax import lax
from jax.experimental import pallas as pl
from jax.experimental.pallas import tpu as pltpu
from jax.experimental.pallas import tpu_sc as plsc

_info = plsc.get_sparse_core_info()
_NC, _NS, _L = _info.num_cores, _info.num_subcores, _info.num_lanes
_NW = _NC * _NS
_B = 16384 * 4
_BPW = _B // _NW
_D = 10
_OPW = _BPW * _D
_PERIOD = 5
_XSTEP = _PERIOD * _L // _D
_NMACRO = _OPW // (_PERIOD * _L)

_mesh = plsc.VectorSubcoreMesh(core_axis_name="c", subcore_axis_name="s")

_PAT = np.asarray(
    [[(i + 16 * g) // _D for i in range(_L)] for g in range(_PERIOD)]
    + [[(i + 16 * g) % _D for i in range(_L)] for g in range(_PERIOD)],
    dtype=np.int32)


@functools.partial(
    pl.kernel,
    mesh=_mesh,
    compiler_params=pltpu.CompilerParams(needs_layout_passes=False),
    out_type=jax.ShapeDtypeStruct((_B * _D,), jnp.float32),
    scratch_types=[
        pltpu.VMEM((_BPW,), jnp.int32),
        pltpu.VMEM((100 * _D,), jnp.float32),
        pltpu.VMEM((_OPW,), jnp.float32),
        pltpu.VMEM((2 * _PERIOD, _L), jnp.int32),
    ],
)
def _gather_kernel(x_hbm, table_hbm, pat_hbm, out_hbm, idx_v, table_v,
                   out_v, pat_v):
    wid = lax.axis_index("s") * _NC + lax.axis_index("c")
    base = wid * _BPW
    pltpu.sync_copy(x_hbm.at[pl.ds(base, _BPW)], idx_v)
    pltpu.sync_copy(table_hbm, table_v)
    pltpu.sync_copy(pat_hbm, pat_v)

    qs = [pat_v[g] for g in range(_PERIOD)]
    rs = [pat_v[_PERIOD + g] for g in range(_PERIOD)]

    @plsc.parallel_loop(0, _NMACRO, unroll=8)
    def _(m):
        xoff = m * _XSTEP
        ooff = m * (_PERIOD * _L)
        for g in range(_PERIOD):
            rows = plsc.load_gather(idx_v, [qs[g] + xoff])
            vals = plsc.load_gather(table_v, [rows * _D + rs[g]])
            out_v[pl.ds(ooff + 16 * g, _L)] = vals
    pltpu.sync_copy(out_v, out_hbm.at[pl.ds(wid * _OPW, _OPW)])


def kernel(x, table):
    out = _gather_kernel(x.reshape(-1), table.reshape(-1),
                         jnp.asarray(_PAT))
    return out.reshape(-1, 2, 2, 10)
